# Initial kernel scaffold; baseline (speedup 1.0000x reference)
import jax
import jax.numpy as jnp
from jax.experimental import pallas as pl


def kernel(CellX, CellEdgeIndex, Wq1, bq1, Wk1, bk1, Wv1, bv1, Ws1, bs1, Wq2, bq2, Wk2, bk2, Wv2, bv2, Ws2, bs2, Wl, bl, Wb, bb):
    raise NotImplementedError("write your pallas kernel here")


# speedup vs baseline: 1.0882x; 1.0882x over previous
import functools

import jax
import jax.numpy as jnp
import numpy as np
from jax.experimental import pallas as pl
from jax.experimental.pallas import tpu as pltpu


def _proj_body(x_ref, w_ref, b_ref, o_ref):
    o_ref[...] = (
        jnp.dot(x_ref[...], w_ref[...], preferred_element_type=jnp.float32)
        + b_ref[...]
    )


def _proj(x, wcat, bcat, block_n=1000):
    n, k = x.shape
    m = wcat.shape[1]
    return pl.pallas_call(
        _proj_body,
        grid=(n // block_n,),
        in_specs=[
            pl.BlockSpec((block_n, k), lambda i: (i, 0)),
            pl.BlockSpec((k, m), lambda i: (0, 0)),
            pl.BlockSpec((1, m), lambda i: (0, 0)),
        ],
        out_specs=pl.BlockSpec((block_n, m), lambda i: (i, 0)),
        out_shape=jax.ShapeDtypeStruct((n, m), jnp.float32),
    )(x, wcat, bcat)


def _dec_body(hh_ref, w_ref, b_ref, o_ref):
    acc = jnp.dot(hh_ref[...], w_ref[...], preferred_element_type=jnp.float32)
    o_ref[...] = jax.nn.sigmoid(acc + b_ref[...])


def _decode(hh, w2, bb, block_n=1000):
    n, aa = hh.shape
    g = w2.shape[1]
    return pl.pallas_call(
        _dec_body,
        grid=(n // block_n,),
        in_specs=[
            pl.BlockSpec((block_n, aa), lambda i: (i, 0)),
            pl.BlockSpec((aa, g), lambda i: (0, 0)),
            pl.BlockSpec((1, g), lambda i: (0, 0)),
        ],
        out_specs=pl.BlockSpec((block_n, g), lambda i: (i, 0)),
        out_shape=jax.ShapeDtypeStruct((n, g), jnp.float32),
    )(hh, w2, bb)


def _edge_attention(q, k, v, src, dst, n, d):
    logits = jnp.sum(q[dst] * k[src], axis=-1) / np.sqrt(d)
    m = jax.ops.segment_max(logits, dst, num_segments=n)
    m = jnp.where(jnp.isfinite(m), m, 0.0)
    a = jnp.exp(logits - m[dst])
    denom = jax.ops.segment_sum(a, dst, num_segments=n)
    agg = jax.ops.segment_sum(v[src] * a[:, None], dst, num_segments=n)
    return agg / (denom[:, None] + 1e-16)


def kernel(CellX, CellEdgeIndex, Wq1, bq1, Wk1, bk1, Wv1, bv1, Ws1, bs1,
           Wq2, bq2, Wk2, bk2, Wv2, bv2, Ws2, bs2, Wl, bl, Wb, bb):
    n = CellX.shape[0]
    src = CellEdgeIndex[0]
    dst = CellEdgeIndex[1]
    d1 = Wq1.shape[0]
    d2 = Wq2.shape[0]

    wcat1 = jnp.concatenate([Wq1, Wk1, Wv1, Ws1], axis=0).T
    bcat1 = jnp.concatenate([bq1, bk1, bv1, bs1])[None, :]
    proj1 = _proj(CellX, wcat1, bcat1)
    q1, k1, v1, s1 = (proj1[:, :d1], proj1[:, d1:2 * d1],
                      proj1[:, 2 * d1:3 * d1], proj1[:, 3 * d1:])
    h1 = jax.nn.relu(_edge_attention(q1, k1, v1, src, dst, n, d1) + s1)

    wcat2 = jnp.concatenate([Wq2, Wk2, Wv2, Ws2], axis=0).T
    bcat2 = jnp.concatenate([bq2, bk2, bv2, bs2])[None, :]
    proj2 = _proj(h1, wcat2, bcat2)
    q2, k2, v2, s2 = (proj2[:, :d2], proj2[:, d2:2 * d2],
                      proj2[:, 2 * d2:3 * d2], proj2[:, 3 * d2:])
    z_mean = _edge_attention(q2, k2, v2, src, dst, n, d2) + s2

    h = z_mean @ Wl.T + bl
    adj = Wl.shape[0]
    hh = (h[:, :, None] * h[:, None, :]).reshape(n, adj * adj)
    w2 = Wb.transpose(1, 2, 0).reshape(adj * adj, Wb.shape[0])
    dec = _decode(hh, w2, bb[None, :])
    return (dec, z_mean)


# speedup vs baseline: 7.8887x; 7.2492x over previous
import functools

import jax
import jax.numpy as jnp
import numpy as np
from jax import lax
from jax.experimental import pallas as pl
from jax.experimental.pallas import tpu as pltpu
from jax.experimental.pallas import tpu_sc as plsc

NC, NS = 2, 16
NW = NC * NS
EDGE_CHUNK = 80

_REP_NP = np.zeros((32, 1024), np.float32)
_TILE_NP = np.zeros((32, 1024), np.float32)
for _i in range(32):
    _REP_NP[_i, 32 * _i:32 * _i + 32] = 1.0
    _TILE_NP[_i, _i::32] = 1.0


def _proj1_body(x_ref, w_ref, b_ref, oq, ok, ov, os_, oqn, okn):
    t = jnp.dot(x_ref[...], w_ref[...], preferred_element_type=jnp.float32)
    t = t + b_ref[...]
    q = t[:, 0:128]
    k = t[:, 128:256]
    oq[...] = q
    ok[...] = k
    ov[...] = t[:, 256:384]
    os_[...] = t[:, 384:512]
    oqn[...] = jnp.sum(q * q, axis=1, keepdims=True)
    okn[...] = jnp.sum(k * k, axis=1, keepdims=True)


def _proj2_body(a0_ref, a1_ref, dd_ref, s1_ref, w_ref, b_ref,
                oq, ok, ov, os_, oqn, okn):
    agg = a0_ref[...] + a1_ref[...]
    den = jnp.sum(dd_ref[...], axis=1, keepdims=True)
    h1 = jax.nn.relu(agg / (den + 1e-16) + s1_ref[...])
    t = jnp.dot(h1, w_ref[...], preferred_element_type=jnp.float32)
    t = t + b_ref[...]
    q = t[:, 0:16]
    k = t[:, 16:32]
    oq[...] = q
    ok[...] = k
    ov[...] = t[:, 32:48]
    os_[...] = t[:, 48:64]
    oqn[...] = jnp.sum(q * q, axis=1, keepdims=True)
    okn[...] = jnp.sum(k * k, axis=1, keepdims=True)


def _decode_body(a0_ref, a1_ref, s2_ref, wl_ref, bl_ref, rep_ref, til_ref,
                 w2_ref, bb_ref, odec, oz):
    agg = a0_ref[:, 0:16] + a1_ref[:, 0:16]
    den = a0_ref[:, 16:17] + a1_ref[:, 16:17]
    z = agg / (den + 1e-16) + s2_ref[...]
    oz[...] = z
    h = jnp.dot(z, wl_ref[...], preferred_element_type=jnp.float32)
    h = h + bl_ref[...]
    hh = (jnp.dot(h, rep_ref[...], preferred_element_type=jnp.float32)
          * jnp.dot(h, til_ref[...], preferred_element_type=jnp.float32))
    acc = jnp.dot(hh, w2_ref[...], preferred_element_type=jnp.float32)
    odec[...] = jax.nn.sigmoid(acc + bb_ref[...])


def _make_edge128(n, e):
    d = 128
    c_edges = EDGE_CHUNK
    e_per_w = e // NW
    chunks = e_per_w // c_edges
    assert chunks * c_edges * NW == e
    n_pad = -(-n // (NS * 8)) * (NS * 8)
    rows_per_tile = n_pad // NS
    nd = d // 16
    mesh = plsc.VectorSubcoreMesh(core_axis_name="c", subcore_axis_name="s")

    @functools.partial(
        pl.kernel,
        mesh=mesh,
        out_type=(jax.ShapeDtypeStruct((NC, n_pad, d), jnp.float32),
                  jax.ShapeDtypeStruct((NC, NS, n_pad), jnp.float32)),
        compiler_params=pltpu.CompilerParams(use_tc_tiling_on_sc=False,
                                             needs_layout_passes=False),
        scratch_types=[
            pltpu.VMEM((c_edges,), jnp.int32),
            pltpu.VMEM((c_edges,), jnp.int32),
            pltpu.VMEM((c_edges, d), jnp.float32),
            pltpu.VMEM((c_edges, d), jnp.float32),
            pltpu.VMEM((c_edges, d), jnp.float32),
            pltpu.VMEM((c_edges, 16), jnp.float32),
            pltpu.VMEM((n_pad,), jnp.float32),
            pltpu.VMEM((16,), jnp.float32),
            pltpu.VMEM_SHARED((n_pad, d), jnp.float32),
            pltpu.SemaphoreType.DMA,
            pltpu.SemaphoreType.DMA,
            pltpu.SemaphoreType.DMA,
        ],
    )
    def edge_kernel(q_hbm, k_hbm, v_hbm, src_hbm, dst_hbm, m_hbm, z_hbm,
                    outv_hbm, outd_hbm, src_v, dst_v, qr, kr, vr, a_st,
                    dden, mv, acc, sm1, sm2, sm3):
        core = lax.axis_index("c")
        sub = lax.axis_index("s")
        wid = sub * NC + core
        r0 = sub * rows_per_tile
        rows = pl.ds(r0, rows_per_tile)
        pltpu.sync_copy(z_hbm.at[rows], acc.at[rows])
        pltpu.sync_copy(m_hbm, mv)
        zero16 = jnp.zeros((16,), jnp.float32)

        def zinit(i, carry):
            dden[pl.ds(i * 16, 16)] = zero16
            return carry

        lax.fori_loop(0, n_pad // 16, zinit, 0)
        plsc.subcore_barrier()

        e0 = wid * e_per_w

        def chunk(i, carry):
            base = e0 + i * c_edges
            pltpu.sync_copy(src_hbm.at[pl.ds(base, c_edges)], src_v)
            pltpu.sync_copy(dst_hbm.at[pl.ds(base, c_edges)], dst_v)
            cp1 = pltpu.async_copy(q_hbm.at[dst_v], qr, sm1)
            cp2 = pltpu.async_copy(k_hbm.at[src_v], kr, sm2)
            cp3 = pltpu.async_copy(v_hbm.at[src_v], vr, sm3)
            cp1.wait()
            cp2.wait()
            cp3.wait()
            mvec = mv[...]

            def per_edge(ei, carry2):
                accv = qr[ei, pl.ds(0, 16)] * kr[ei, pl.ds(0, 16)]
                for j in range(1, nd):
                    accv = accv + qr[ei, pl.ds(16 * j, 16)] * kr[ei, pl.ds(16 * j, 16)]
                logit = jnp.sum(accv)
                avec = jnp.exp(jnp.full((16,), logit) - mvec)
                for j in range(nd):
                    vr[ei, pl.ds(16 * j, 16)] = vr[ei, pl.ds(16 * j, 16)] * avec
                a_st[ei, pl.ds(0, 16)] = avec
                return carry2

            lax.fori_loop(0, c_edges, per_edge, 0)
            col0 = jnp.zeros((16,), jnp.int32)
            for g in range(c_edges // 16):
                rid = jnp.arange(16, dtype=jnp.int32) + (16 * g)
                a16 = plsc.load_gather(a_st, [rid, col0])
                idx16 = dst_v[pl.ds(16 * g, 16)]
                plsc.addupdate_scatter(dden, [idx16], a16)
            pltpu.sync_copy(vr, acc.at[dst_v], add=True)
            return carry

        lax.fori_loop(0, chunks, chunk, 0)
        plsc.subcore_barrier()
        pltpu.sync_copy(acc.at[rows], outv_hbm.at[core, rows])
        pltpu.sync_copy(dden, outd_hbm.at[core, sub])

    return edge_kernel


def _make_edge16(n, e):
    d = 16
    av_w = 32
    c_edges = EDGE_CHUNK
    e_per_w = e // NW
    chunks = e_per_w // c_edges
    n_pad = -(-n // (NS * 8)) * (NS * 8)
    rows_per_tile = n_pad // NS
    mesh = plsc.VectorSubcoreMesh(core_axis_name="c", subcore_axis_name="s")

    @functools.partial(
        pl.kernel,
        mesh=mesh,
        out_type=jax.ShapeDtypeStruct((NC, n_pad, av_w), jnp.float32),
        compiler_params=pltpu.CompilerParams(use_tc_tiling_on_sc=False,
                                             needs_layout_passes=False),
        scratch_types=[
            pltpu.VMEM((c_edges,), jnp.int32),
            pltpu.VMEM((c_edges,), jnp.int32),
            pltpu.VMEM((c_edges, d), jnp.float32),
            pltpu.VMEM((c_edges, d), jnp.float32),
            pltpu.VMEM((c_edges, d), jnp.float32),
            pltpu.VMEM((c_edges, av_w), jnp.float32),
            pltpu.VMEM((16,), jnp.float32),
            pltpu.VMEM_SHARED((n_pad, av_w), jnp.float32),
            pltpu.SemaphoreType.DMA,
            pltpu.SemaphoreType.DMA,
            pltpu.SemaphoreType.DMA,
        ],
    )
    def edge_kernel(q_hbm, k_hbm, v_hbm, src_hbm, dst_hbm, m_hbm, z_hbm,
                    out_hbm, src_v, dst_v, qr, kr, vr, av, mv, acc,
                    sm1, sm2, sm3):
        core = lax.axis_index("c")
        sub = lax.axis_index("s")
        wid = sub * NC + core
        r0 = sub * rows_per_tile
        rows = pl.ds(r0, rows_per_tile)
        pltpu.sync_copy(z_hbm.at[rows], acc.at[rows])
        pltpu.sync_copy(m_hbm, mv)
        plsc.subcore_barrier()

        e0 = wid * e_per_w

        def chunk(i, carry):
            base = e0 + i * c_edges
            pltpu.sync_copy(src_hbm.at[pl.ds(base, c_edges)], src_v)
            pltpu.sync_copy(dst_hbm.at[pl.ds(base, c_edges)], dst_v)
            cp1 = pltpu.async_copy(q_hbm.at[dst_v], qr, sm1)
            cp2 = pltpu.async_copy(k_hbm.at[src_v], kr, sm2)
            cp3 = pltpu.async_copy(v_hbm.at[src_v], vr, sm3)
            cp1.wait()
            cp2.wait()
            cp3.wait()
            mvec = mv[...]

            def per_edge(ei, carry2):
                accv = qr[ei, pl.ds(0, 16)] * kr[ei, pl.ds(0, 16)]
                logit = jnp.sum(accv)
                avec = jnp.exp(jnp.full((16,), logit) - mvec)
                av[ei, pl.ds(0, 16)] = vr[ei, pl.ds(0, 16)] * avec
                av[ei, pl.ds(16, 16)] = avec
                return carry2

            lax.fori_loop(0, c_edges, per_edge, 0)
            pltpu.sync_copy(av, acc.at[dst_v], add=True)
            return carry

        lax.fori_loop(0, chunks, chunk, 0)
        plsc.subcore_barrier()
        pltpu.sync_copy(acc.at[rows], out_hbm.at[core, rows])

    return edge_kernel


def _proj1(x, wcat, bcat, block_n=1000):
    n, kdim = x.shape
    m = wcat.shape[1]
    d = 128
    return pl.pallas_call(
        _proj1_body,
        grid=(n // block_n,),
        in_specs=[
            pl.BlockSpec((block_n, kdim), lambda i: (i, 0)),
            pl.BlockSpec((kdim, m), lambda i: (0, 0)),
            pl.BlockSpec((1, m), lambda i: (0, 0)),
        ],
        out_specs=[
            pl.BlockSpec((block_n, d), lambda i: (i, 0)),
            pl.BlockSpec((block_n, d), lambda i: (i, 0)),
            pl.BlockSpec((block_n, d), lambda i: (i, 0)),
            pl.BlockSpec((block_n, d), lambda i: (i, 0)),
            pl.BlockSpec((block_n, 1), lambda i: (i, 0)),
            pl.BlockSpec((block_n, 1), lambda i: (i, 0)),
        ],
        out_shape=[
            jax.ShapeDtypeStruct((n, d), jnp.float32),
            jax.ShapeDtypeStruct((n, d), jnp.float32),
            jax.ShapeDtypeStruct((n, d), jnp.float32),
            jax.ShapeDtypeStruct((n, d), jnp.float32),
            jax.ShapeDtypeStruct((n, 1), jnp.float32),
            jax.ShapeDtypeStruct((n, 1), jnp.float32),
        ],
    )(x, wcat, bcat)


def _proj2(a0, a1, dd, s1, wcat, bcat, block_n=1000):
    n = a0.shape[0]
    aw = a0.shape[1]
    m = wcat.shape[1]
    d = 16
    return pl.pallas_call(
        _proj2_body,
        grid=(n // block_n,),
        in_specs=[
            pl.BlockSpec((block_n, aw), lambda i: (i, 0)),
            pl.BlockSpec((block_n, aw), lambda i: (i, 0)),
            pl.BlockSpec((block_n, NW), lambda i: (i, 0)),
            pl.BlockSpec((block_n, 128), lambda i: (i, 0)),
            pl.BlockSpec((128, m), lambda i: (0, 0)),
            pl.BlockSpec((1, m), lambda i: (0, 0)),
        ],
        out_specs=[
            pl.BlockSpec((block_n, d), lambda i: (i, 0)),
            pl.BlockSpec((block_n, d), lambda i: (i, 0)),
            pl.BlockSpec((block_n, d), lambda i: (i, 0)),
            pl.BlockSpec((block_n, d), lambda i: (i, 0)),
            pl.BlockSpec((block_n, 1), lambda i: (i, 0)),
            pl.BlockSpec((block_n, 1), lambda i: (i, 0)),
        ],
        out_shape=[
            jax.ShapeDtypeStruct((n, d), jnp.float32),
            jax.ShapeDtypeStruct((n, d), jnp.float32),
            jax.ShapeDtypeStruct((n, d), jnp.float32),
            jax.ShapeDtypeStruct((n, d), jnp.float32),
            jax.ShapeDtypeStruct((n, 1), jnp.float32),
            jax.ShapeDtypeStruct((n, 1), jnp.float32),
        ],
    )(a0, a1, dd, s1, wcat, bcat)


def _decode(a0, a1, s2, wl, bl, w2, bb, block_n=1000):
    n = a0.shape[0]
    aw = a0.shape[1]
    g = w2.shape[1]
    rep = jnp.asarray(_REP_NP)
    til = jnp.asarray(_TILE_NP)
    return pl.pallas_call(
        _decode_body,
        grid=(n // block_n,),
        in_specs=[
            pl.BlockSpec((block_n, aw), lambda i: (i, 0)),
            pl.BlockSpec((block_n, aw), lambda i: (i, 0)),
            pl.BlockSpec((block_n, 16), lambda i: (i, 0)),
            pl.BlockSpec((16, 32), lambda i: (0, 0)),
            pl.BlockSpec((1, 32), lambda i: (0, 0)),
            pl.BlockSpec((32, 1024), lambda i: (0, 0)),
            pl.BlockSpec((32, 1024), lambda i: (0, 0)),
            pl.BlockSpec((1024, g), lambda i: (0, 0)),
            pl.BlockSpec((1, g), lambda i: (0, 0)),
        ],
        out_specs=[
            pl.BlockSpec((block_n, g), lambda i: (i, 0)),
            pl.BlockSpec((block_n, 16), lambda i: (i, 0)),
        ],
        out_shape=[
            jax.ShapeDtypeStruct((n, g), jnp.float32),
            jax.ShapeDtypeStruct((n, 16), jnp.float32),
        ],
    )(a0, a1, s2, wl, bl, rep, til, w2, bb)


def _pad_cols(w, to):
    return jnp.pad(w, ((0, 0), (0, to - w.shape[1])))


def kernel(CellX, CellEdgeIndex, Wq1, bq1, Wk1, bk1, Wv1, bv1, Ws1, bs1,
           Wq2, bq2, Wk2, bk2, Wv2, bv2, Ws2, bs2, Wl, bl, Wb, bb):
    n = CellX.shape[0]
    e = CellEdgeIndex.shape[1]
    src = CellEdgeIndex[0]
    dst = CellEdgeIndex[1]
    d1 = Wq1.shape[0]
    d2 = Wq2.shape[0]
    gene = Wb.shape[0]
    adj = Wl.shape[0]

    inv1 = 1.0 / np.sqrt(d1)
    wcat1 = jnp.concatenate([Wq1 * inv1, Wk1, Wv1, Ws1], axis=0).T
    bcat1 = jnp.concatenate([bq1 * inv1, bk1, bv1, bs1])[None, :]
    q1, k1, v1, s1, qn1, kn1 = _proj1(CellX, wcat1, bcat1)
    m1 = jnp.sqrt(jnp.max(qn1) * jnp.max(kn1))
    mv1 = jnp.full((16,), m1, jnp.float32)

    n_pad = -(-n // (NS * 8)) * (NS * 8)
    ek1 = _make_edge128(n, e)
    acc1, den1 = ek1(q1, k1, v1, src, dst, mv1,
                     jnp.zeros((n_pad, d1), jnp.float32))
    dd1 = den1.reshape(NW, n_pad).T

    inv2 = 1.0 / np.sqrt(d2)
    wcat2 = jnp.concatenate(
        [_pad_cols((Wq2 * inv2).T, 16), _pad_cols(Wk2.T, 16),
         _pad_cols(Wv2.T, 16), _pad_cols(Ws2.T, 16)], axis=1)
    bcat2 = jnp.concatenate(
        [jnp.pad(bq2 * inv2, (0, 1)), jnp.pad(bk2, (0, 1)),
         jnp.pad(bv2, (0, 1)), jnp.pad(bs2, (0, 1))])[None, :]
    q2, k2, v2, s2, qn2, kn2 = _proj2(acc1[0, :n], acc1[1, :n], dd1[:n],
                                      s1, wcat2, bcat2)
    m2 = jnp.sqrt(jnp.max(qn2) * jnp.max(kn2))
    mv2 = jnp.full((16,), m2, jnp.float32)

    ek2 = _make_edge16(n, e)
    acc2 = ek2(q2, k2, v2, src, dst, mv2,
               jnp.zeros((n_pad, 32), jnp.float32))

    wl_pad = jnp.pad(Wl.T, ((0, 1), (0, 0)))
    w2 = Wb.transpose(1, 2, 0).reshape(adj * adj, gene)
    dec, z_pad = _decode(acc2[0, :n], acc2[1, :n], s2, wl_pad, bl[None, :],
                         w2, bb[None, :])
    return (dec, z_pad[:, :d2])


# speedup vs baseline: 10.8509x; 1.3755x over previous
import functools

import jax
import jax.numpy as jnp
import numpy as np
from jax import lax
from jax.experimental import pallas as pl
from jax.experimental.pallas import tpu as pltpu
from jax.experimental.pallas import tpu_sc as plsc

NC, NS = 2, 16
NW = NC * NS
EDGE_CHUNK = 80

_REP_NP = np.zeros((32, 1024), np.float32)
_TILE_NP = np.zeros((32, 1024), np.float32)
for _i in range(32):
    _REP_NP[_i, 32 * _i:32 * _i + 32] = 1.0
    _TILE_NP[_i, _i::32] = 1.0


def _proj1_body(x_ref, w_ref, b_ref, oq, ok, ov, os_, oqn, okn):
    t = jnp.dot(x_ref[...], w_ref[...], preferred_element_type=jnp.float32)
    t = t + b_ref[...]
    q = t[:, 0:128]
    k = t[:, 128:256]
    oq[...] = q
    ok[...] = k
    ov[...] = t[:, 256:384]
    os_[...] = t[:, 384:512]
    oqn[...] = jnp.sum(q * q, axis=1, keepdims=True)
    okn[...] = jnp.sum(k * k, axis=1, keepdims=True)


def _proj2_body(a0_ref, a1_ref, dd_ref, s1_ref, w_ref, b_ref,
                oq, ok, ov, os_, oqn, okn):
    agg = a0_ref[...] + a1_ref[...]
    den = jnp.sum(dd_ref[...], axis=1, keepdims=True)
    h1 = jax.nn.relu(agg / (den + 1e-16) + s1_ref[...])
    t = jnp.dot(h1, w_ref[...], preferred_element_type=jnp.float32)
    t = t + b_ref[...]
    q = t[:, 0:16]
    k = t[:, 16:32]
    oq[...] = q
    ok[...] = k
    ov[...] = t[:, 32:48]
    os_[...] = t[:, 48:64]
    oqn[...] = jnp.sum(q * q, axis=1, keepdims=True)
    okn[...] = jnp.sum(k * k, axis=1, keepdims=True)


def _decode_body(a0_ref, a1_ref, s2_ref, wl_ref, bl_ref, rep_ref, til_ref,
                 w2_ref, bb_ref, odec, oz):
    agg = a0_ref[:, 0:16] + a1_ref[:, 0:16]
    den = a0_ref[:, 16:17] + a1_ref[:, 16:17]
    z = agg / (den + 1e-16) + s2_ref[...]
    oz[...] = z
    h = jnp.dot(z, wl_ref[...], preferred_element_type=jnp.float32)
    h = h + bl_ref[...]
    hh = (jnp.dot(h, rep_ref[...], preferred_element_type=jnp.float32)
          * jnp.dot(h, til_ref[...], preferred_element_type=jnp.float32))
    acc = jnp.dot(hh, w2_ref[...], preferred_element_type=jnp.float32)
    odec[...] = jax.nn.sigmoid(acc + bb_ref[...])


def _make_edge128(n, e):
    d = 128
    c_edges = EDGE_CHUNK
    e_per_w = e // NW
    chunks = e_per_w // c_edges
    assert chunks * c_edges * NW == e
    n_pad = -(-n // (NS * 8)) * (NS * 8)
    rows_per_tile = n_pad // NS
    nd = d // 16
    mesh = plsc.VectorSubcoreMesh(core_axis_name="c", subcore_axis_name="s")

    @functools.partial(
        pl.kernel,
        mesh=mesh,
        out_type=(jax.ShapeDtypeStruct((NC, n_pad, d), jnp.float32),
                  jax.ShapeDtypeStruct((NC, NS, n_pad), jnp.float32)),
        compiler_params=pltpu.CompilerParams(use_tc_tiling_on_sc=False,
                                             needs_layout_passes=False),
        scratch_types=[
            pltpu.VMEM((c_edges,), jnp.int32),
            pltpu.VMEM((c_edges,), jnp.int32),
            pltpu.VMEM((c_edges, d), jnp.float32),
            pltpu.VMEM((c_edges, d), jnp.float32),
            pltpu.VMEM((c_edges, d), jnp.float32),
            pltpu.VMEM((c_edges, 16), jnp.float32),
            pltpu.VMEM((n_pad,), jnp.float32),
            pltpu.VMEM((16,), jnp.float32),
            pltpu.VMEM_SHARED((n_pad, d), jnp.float32),
            pltpu.SemaphoreType.DMA,
            pltpu.SemaphoreType.DMA,
            pltpu.SemaphoreType.DMA,
        ],
    )
    def edge_kernel(q_hbm, k_hbm, v_hbm, src_hbm, dst_hbm, m_hbm, z_hbm,
                    outv_hbm, outd_hbm, src_v, dst_v, qr, kr, vr, a_st,
                    dden, mv, acc, sm1, sm2, sm3):
        core = lax.axis_index("c")
        sub = lax.axis_index("s")
        wid = sub * NC + core
        r0 = sub * rows_per_tile
        rows = pl.ds(r0, rows_per_tile)
        pltpu.sync_copy(z_hbm.at[rows], acc.at[rows])
        pltpu.sync_copy(m_hbm, mv)
        zero16 = jnp.zeros((16,), jnp.float32)

        def zinit(i, carry):
            dden[pl.ds(i * 16, 16)] = zero16
            return carry

        lax.fori_loop(0, n_pad // 16, zinit, 0)
        plsc.subcore_barrier()

        e0 = wid * e_per_w

        def chunk(i, carry):
            base = e0 + i * c_edges
            pltpu.sync_copy(src_hbm.at[pl.ds(base, c_edges)], src_v)
            pltpu.sync_copy(dst_hbm.at[pl.ds(base, c_edges)], dst_v)
            cp1 = pltpu.async_copy(q_hbm.at[dst_v], qr, sm1)
            cp2 = pltpu.async_copy(k_hbm.at[src_v], kr, sm2)
            cp3 = pltpu.async_copy(v_hbm.at[src_v], vr, sm3)
            cp1.wait()
            cp2.wait()
            cp3.wait()
            mvec = mv[...]

            @plsc.parallel_loop(0, c_edges, step=1, unroll=4)
            def _(ei):
                accv = qr[ei, pl.ds(0, 16)] * kr[ei, pl.ds(0, 16)]
                for j in range(1, nd):
                    accv = accv + qr[ei, pl.ds(16 * j, 16)] * kr[ei, pl.ds(16 * j, 16)]
                logit = jnp.sum(accv)
                avec = jnp.exp(jnp.full((16,), logit) - mvec)
                for j in range(nd):
                    vr[ei, pl.ds(16 * j, 16)] = vr[ei, pl.ds(16 * j, 16)] * avec
                a_st[ei, pl.ds(0, 16)] = avec
            col0 = jnp.zeros((16,), jnp.int32)
            for g in range(c_edges // 16):
                rid = jnp.arange(16, dtype=jnp.int32) + (16 * g)
                a16 = plsc.load_gather(a_st, [rid, col0])
                idx16 = dst_v[pl.ds(16 * g, 16)]
                plsc.addupdate_scatter(dden, [idx16], a16)
            pltpu.sync_copy(vr, acc.at[dst_v], add=True)
            return carry

        lax.fori_loop(0, chunks, chunk, 0)
        plsc.subcore_barrier()
        pltpu.sync_copy(acc.at[rows], outv_hbm.at[core, rows])
        pltpu.sync_copy(dden, outd_hbm.at[core, sub])

    return edge_kernel


def _make_edge16(n, e):
    d = 16
    av_w = 32
    c_edges = EDGE_CHUNK
    e_per_w = e // NW
    chunks = e_per_w // c_edges
    n_pad = -(-n // (NS * 8)) * (NS * 8)
    rows_per_tile = n_pad // NS
    mesh = plsc.VectorSubcoreMesh(core_axis_name="c", subcore_axis_name="s")

    @functools.partial(
        pl.kernel,
        mesh=mesh,
        out_type=jax.ShapeDtypeStruct((NC, n_pad, av_w), jnp.float32),
        compiler_params=pltpu.CompilerParams(use_tc_tiling_on_sc=False,
                                             needs_layout_passes=False),
        scratch_types=[
            pltpu.VMEM((c_edges,), jnp.int32),
            pltpu.VMEM((c_edges,), jnp.int32),
            pltpu.VMEM((c_edges, d), jnp.float32),
            pltpu.VMEM((c_edges, d), jnp.float32),
            pltpu.VMEM((c_edges, d), jnp.float32),
            pltpu.VMEM((c_edges, av_w), jnp.float32),
            pltpu.VMEM((16,), jnp.float32),
            pltpu.VMEM_SHARED((n_pad, av_w), jnp.float32),
            pltpu.SemaphoreType.DMA,
            pltpu.SemaphoreType.DMA,
            pltpu.SemaphoreType.DMA,
        ],
    )
    def edge_kernel(q_hbm, k_hbm, v_hbm, src_hbm, dst_hbm, m_hbm, z_hbm,
                    out_hbm, src_v, dst_v, qr, kr, vr, av, mv, acc,
                    sm1, sm2, sm3):
        core = lax.axis_index("c")
        sub = lax.axis_index("s")
        wid = sub * NC + core
        r0 = sub * rows_per_tile
        rows = pl.ds(r0, rows_per_tile)
        pltpu.sync_copy(z_hbm.at[rows], acc.at[rows])
        pltpu.sync_copy(m_hbm, mv)
        plsc.subcore_barrier()

        e0 = wid * e_per_w

        def chunk(i, carry):
            base = e0 + i * c_edges
            pltpu.sync_copy(src_hbm.at[pl.ds(base, c_edges)], src_v)
            pltpu.sync_copy(dst_hbm.at[pl.ds(base, c_edges)], dst_v)
            cp1 = pltpu.async_copy(q_hbm.at[dst_v], qr, sm1)
            cp2 = pltpu.async_copy(k_hbm.at[src_v], kr, sm2)
            cp3 = pltpu.async_copy(v_hbm.at[src_v], vr, sm3)
            cp1.wait()
            cp2.wait()
            cp3.wait()
            mvec = mv[...]

            @plsc.parallel_loop(0, c_edges, step=1, unroll=8)
            def _(ei):
                accv = qr[ei, pl.ds(0, 16)] * kr[ei, pl.ds(0, 16)]
                logit = jnp.sum(accv)
                avec = jnp.exp(jnp.full((16,), logit) - mvec)
                av[ei, pl.ds(0, 16)] = vr[ei, pl.ds(0, 16)] * avec
                av[ei, pl.ds(16, 16)] = avec
            pltpu.sync_copy(av, acc.at[dst_v], add=True)
            return carry

        lax.fori_loop(0, chunks, chunk, 0)
        plsc.subcore_barrier()
        pltpu.sync_copy(acc.at[rows], out_hbm.at[core, rows])

    return edge_kernel


def _proj1(x, wcat, bcat, block_n=1000):
    n, kdim = x.shape
    m = wcat.shape[1]
    d = 128
    return pl.pallas_call(
        _proj1_body,
        grid=(n // block_n,),
        in_specs=[
            pl.BlockSpec((block_n, kdim), lambda i: (i, 0)),
            pl.BlockSpec((kdim, m), lambda i: (0, 0)),
            pl.BlockSpec((1, m), lambda i: (0, 0)),
        ],
        out_specs=[
            pl.BlockSpec((block_n, d), lambda i: (i, 0)),
            pl.BlockSpec((block_n, d), lambda i: (i, 0)),
            pl.BlockSpec((block_n, d), lambda i: (i, 0)),
            pl.BlockSpec((block_n, d), lambda i: (i, 0)),
            pl.BlockSpec((block_n, 1), lambda i: (i, 0)),
            pl.BlockSpec((block_n, 1), lambda i: (i, 0)),
        ],
        out_shape=[
            jax.ShapeDtypeStruct((n, d), jnp.float32),
            jax.ShapeDtypeStruct((n, d), jnp.float32),
            jax.ShapeDtypeStruct((n, d), jnp.float32),
            jax.ShapeDtypeStruct((n, d), jnp.float32),
            jax.ShapeDtypeStruct((n, 1), jnp.float32),
            jax.ShapeDtypeStruct((n, 1), jnp.float32),
        ],
    )(x, wcat, bcat)


def _proj2(a0, a1, dd, s1, wcat, bcat, block_n=1000):
    n = a0.shape[0]
    aw = a0.shape[1]
    m = wcat.shape[1]
    d = 16
    return pl.pallas_call(
        _proj2_body,
        grid=(n // block_n,),
        in_specs=[
            pl.BlockSpec((block_n, aw), lambda i: (i, 0)),
            pl.BlockSpec((block_n, aw), lambda i: (i, 0)),
            pl.BlockSpec((block_n, NW), lambda i: (i, 0)),
            pl.BlockSpec((block_n, 128), lambda i: (i, 0)),
            pl.BlockSpec((128, m), lambda i: (0, 0)),
            pl.BlockSpec((1, m), lambda i: (0, 0)),
        ],
        out_specs=[
            pl.BlockSpec((block_n, d), lambda i: (i, 0)),
            pl.BlockSpec((block_n, d), lambda i: (i, 0)),
            pl.BlockSpec((block_n, d), lambda i: (i, 0)),
            pl.BlockSpec((block_n, d), lambda i: (i, 0)),
            pl.BlockSpec((block_n, 1), lambda i: (i, 0)),
            pl.BlockSpec((block_n, 1), lambda i: (i, 0)),
        ],
        out_shape=[
            jax.ShapeDtypeStruct((n, d), jnp.float32),
            jax.ShapeDtypeStruct((n, d), jnp.float32),
            jax.ShapeDtypeStruct((n, d), jnp.float32),
            jax.ShapeDtypeStruct((n, d), jnp.float32),
            jax.ShapeDtypeStruct((n, 1), jnp.float32),
            jax.ShapeDtypeStruct((n, 1), jnp.float32),
        ],
    )(a0, a1, dd, s1, wcat, bcat)


def _decode(a0, a1, s2, wl, bl, w2, bb, block_n=1000):
    n = a0.shape[0]
    aw = a0.shape[1]
    g = w2.shape[1]
    rep = jnp.asarray(_REP_NP)
    til = jnp.asarray(_TILE_NP)
    return pl.pallas_call(
        _decode_body,
        grid=(n // block_n,),
        in_specs=[
            pl.BlockSpec((block_n, aw), lambda i: (i, 0)),
            pl.BlockSpec((block_n, aw), lambda i: (i, 0)),
            pl.BlockSpec((block_n, 16), lambda i: (i, 0)),
            pl.BlockSpec((16, 32), lambda i: (0, 0)),
            pl.BlockSpec((1, 32), lambda i: (0, 0)),
            pl.BlockSpec((32, 1024), lambda i: (0, 0)),
            pl.BlockSpec((32, 1024), lambda i: (0, 0)),
            pl.BlockSpec((1024, g), lambda i: (0, 0)),
            pl.BlockSpec((1, g), lambda i: (0, 0)),
        ],
        out_specs=[
            pl.BlockSpec((block_n, g), lambda i: (i, 0)),
            pl.BlockSpec((block_n, 16), lambda i: (i, 0)),
        ],
        out_shape=[
            jax.ShapeDtypeStruct((n, g), jnp.float32),
            jax.ShapeDtypeStruct((n, 16), jnp.float32),
        ],
    )(a0, a1, s2, wl, bl, rep, til, w2, bb)


def _pad_cols(w, to):
    return jnp.pad(w, ((0, 0), (0, to - w.shape[1])))


def kernel(CellX, CellEdgeIndex, Wq1, bq1, Wk1, bk1, Wv1, bv1, Ws1, bs1,
           Wq2, bq2, Wk2, bk2, Wv2, bv2, Ws2, bs2, Wl, bl, Wb, bb):
    n = CellX.shape[0]
    e = CellEdgeIndex.shape[1]
    src = CellEdgeIndex[0]
    dst = CellEdgeIndex[1]
    d1 = Wq1.shape[0]
    d2 = Wq2.shape[0]
    gene = Wb.shape[0]
    adj = Wl.shape[0]

    inv1 = 1.0 / np.sqrt(d1)
    wcat1 = jnp.concatenate([Wq1 * inv1, Wk1, Wv1, Ws1], axis=0).T
    bcat1 = jnp.concatenate([bq1 * inv1, bk1, bv1, bs1])[None, :]
    q1, k1, v1, s1, qn1, kn1 = _proj1(CellX, wcat1, bcat1)
    m1 = jnp.sqrt(jnp.max(qn1) * jnp.max(kn1))
    mv1 = jnp.full((16,), m1, jnp.float32)

    n_pad = -(-n // (NS * 8)) * (NS * 8)
    ek1 = _make_edge128(n, e)
    acc1, den1 = ek1(q1, k1, v1, src, dst, mv1,
                     jnp.zeros((n_pad, d1), jnp.float32))
    dd1 = den1.reshape(NW, n_pad).T

    inv2 = 1.0 / np.sqrt(d2)
    wcat2 = jnp.concatenate(
        [_pad_cols((Wq2 * inv2).T, 16), _pad_cols(Wk2.T, 16),
         _pad_cols(Wv2.T, 16), _pad_cols(Ws2.T, 16)], axis=1)
    bcat2 = jnp.concatenate(
        [jnp.pad(bq2 * inv2, (0, 1)), jnp.pad(bk2, (0, 1)),
         jnp.pad(bv2, (0, 1)), jnp.pad(bs2, (0, 1))])[None, :]
    q2, k2, v2, s2, qn2, kn2 = _proj2(acc1[0, :n], acc1[1, :n], dd1[:n],
                                      s1, wcat2, bcat2)
    m2 = jnp.sqrt(jnp.max(qn2) * jnp.max(kn2))
    mv2 = jnp.full((16,), m2, jnp.float32)

    ek2 = _make_edge16(n, e)
    acc2 = ek2(q2, k2, v2, src, dst, mv2,
               jnp.zeros((n_pad, 32), jnp.float32))

    wl_pad = jnp.pad(Wl.T, ((0, 1), (0, 0)))
    w2 = Wb.transpose(1, 2, 0).reshape(adj * adj, gene)
    dec, z_pad = _decode(acc2[0, :n], acc2[1, :n], s2, wl_pad, bl[None, :],
                         w2, bb[None, :])
    return (dec, z_pad[:, :d2])


# speedup vs baseline: 12.9571x; 1.1941x over previous
import functools

import jax
import jax.numpy as jnp
import numpy as np
from jax import lax
from jax.experimental import pallas as pl
from jax.experimental.pallas import tpu as pltpu
from jax.experimental.pallas import tpu_sc as plsc

NC, NS = 2, 16
NW = NC * NS

_REP_NP = np.zeros((32, 1024), np.float32)
_TILE_NP = np.zeros((32, 1024), np.float32)
for _i in range(32):
    _REP_NP[_i, 32 * _i:32 * _i + 32] = 1.0
    _TILE_NP[_i, _i::32] = 1.0


def _proj1_body(x_ref, w_ref, b_ref, oq, okv, os_, oqn, okn):
    t = jnp.dot(x_ref[...], w_ref[...], preferred_element_type=jnp.float32)
    t = t + b_ref[...]
    q = t[:, 0:128]
    k = t[:, 128:256]
    oq[...] = q
    okv[...] = t[:, 128:384]
    os_[...] = t[:, 384:512]
    oqn[...] = jnp.sum(q * q, axis=1, keepdims=True)
    okn[...] = jnp.sum(k * k, axis=1, keepdims=True)


def _proj2_body(a0_ref, a1_ref, s1_ref, w_ref, b_ref,
                oq, okv, os_, oqn, okn):
    agg = a0_ref[:, 0:128] + a1_ref[:, 0:128]
    den = a0_ref[:, 128:129] + a1_ref[:, 128:129]
    h1 = jax.nn.relu(agg / (den + 1e-16) + s1_ref[...])
    t = jnp.dot(h1, w_ref[...], preferred_element_type=jnp.float32)
    t = t + b_ref[...]
    q = t[:, 0:16]
    k = t[:, 16:32]
    oq[...] = q
    okv[...] = t[:, 16:48]
    os_[...] = t[:, 48:64]
    oqn[...] = jnp.sum(q * q, axis=1, keepdims=True)
    okn[...] = jnp.sum(k * k, axis=1, keepdims=True)


def _decode_body(a0_ref, a1_ref, s2_ref, wl_ref, bl_ref, rep_ref, til_ref,
                 w2_ref, bb_ref, odec, oz):
    agg = a0_ref[:, 0:16] + a1_ref[:, 0:16]
    den = a0_ref[:, 16:17] + a1_ref[:, 16:17]
    z = agg / (den + 1e-16) + s2_ref[...]
    oz[...] = z
    h = jnp.dot(z, wl_ref[...], preferred_element_type=jnp.float32)
    h = h + bl_ref[...]
    hh = (jnp.dot(h, rep_ref[...], preferred_element_type=jnp.float32)
          * jnp.dot(h, til_ref[...], preferred_element_type=jnp.float32))
    acc = jnp.dot(hh, w2_ref[...], preferred_element_type=jnp.float32)
    odec[...] = jax.nn.sigmoid(acc + bb_ref[...])


def _make_edge(n, e, d, c_edges, unroll):
    av_w = d + 16
    nd = d // 16
    e_per_w = e // NW
    chunks = e_per_w // c_edges
    assert chunks * c_edges * NW == e and chunks >= 3
    pairs_full = (chunks - 1) // 2
    rem = chunks - 2 * pairs_full
    n_pad = -(-n // (NS * 8)) * (NS * 8)
    rows_per_tile = n_pad // NS
    mesh = plsc.VectorSubcoreMesh(core_axis_name="c", subcore_axis_name="s")

    @functools.partial(
        pl.kernel,
        mesh=mesh,
        out_type=jax.ShapeDtypeStruct((NC, n_pad, av_w), jnp.float32),
        compiler_params=pltpu.CompilerParams(use_tc_tiling_on_sc=False,
                                             needs_layout_passes=False),
        scratch_types=[
            pltpu.VMEM((c_edges,), jnp.int32),
            pltpu.VMEM((c_edges,), jnp.int32),
            pltpu.VMEM((c_edges,), jnp.int32),
            pltpu.VMEM((c_edges,), jnp.int32),
            pltpu.VMEM((c_edges, d), jnp.float32),
            pltpu.VMEM((c_edges, d), jnp.float32),
            pltpu.VMEM((c_edges, 2 * d), jnp.float32),
            pltpu.VMEM((c_edges, 2 * d), jnp.float32),
            pltpu.VMEM((c_edges, av_w), jnp.float32),
            pltpu.VMEM((16,), jnp.float32),
            pltpu.VMEM_SHARED((n_pad, av_w), jnp.float32),
            pltpu.SemaphoreType.DMA,
            pltpu.SemaphoreType.DMA,
        ],
    )
    def edge_kernel(q_hbm, kv_hbm, src_hbm, dst_hbm, m_hbm, z_hbm, out_hbm,
                    srcv0, dstv0, srcv1, dstv1, qr0, qr1, kvr0, kvr1,
                    av, mv, acc, sem0, sem1):
        core = lax.axis_index("c")
        sub = lax.axis_index("s")
        wid = sub * NC + core
        r0 = sub * rows_per_tile
        rows = pl.ds(r0, rows_per_tile)
        pltpu.sync_copy(z_hbm.at[rows], acc.at[rows])
        pltpu.sync_copy(m_hbm, mv)
        plsc.subcore_barrier()

        e0 = wid * e_per_w
        sets = ((srcv0, dstv0, qr0, kvr0, sem0),
                (srcv1, dstv1, qr1, kvr1, sem1))

        def fire(ic, s):
            srcv, dstv, qr, kvr, sem = s
            base = e0 + ic * c_edges
            pltpu.sync_copy(src_hbm.at[pl.ds(base, c_edges)], srcv)
            pltpu.sync_copy(dst_hbm.at[pl.ds(base, c_edges)], dstv)
            pltpu.async_copy(q_hbm.at[dstv], qr, sem)
            pltpu.async_copy(kv_hbm.at[srcv], kvr, sem)

        def drain_compute(s):
            srcv, dstv, qr, kvr, sem = s
            pltpu.make_async_copy(q_hbm.at[dstv], qr, sem).wait()
            pltpu.make_async_copy(kv_hbm.at[srcv], kvr, sem).wait()
            mvec = mv[...]

            @plsc.parallel_loop(0, c_edges, step=1, unroll=unroll)
            def _(ei):
                accv = qr[ei, pl.ds(0, 16)] * kvr[ei, pl.ds(0, 16)]
                for j in range(1, nd):
                    accv = accv + (qr[ei, pl.ds(16 * j, 16)]
                                   * kvr[ei, pl.ds(16 * j, 16)])
                logit = jnp.sum(accv)
                avec = jnp.exp(jnp.full((16,), logit) - mvec)
                for j in range(nd):
                    av[ei, pl.ds(16 * j, 16)] = (
                        kvr[ei, pl.ds(d + 16 * j, 16)] * avec)
                av[ei, pl.ds(d, 16)] = avec

            pltpu.sync_copy(av, acc.at[dstv], add=True)

        fire(0, sets[0])

        def pair(i, carry):
            for b in range(2):
                fire(2 * i + b + 1, sets[1 - b])
                drain_compute(sets[b])
            return carry

        lax.fori_loop(0, pairs_full, pair, 0)
        if rem == 2:
            fire(chunks - 1, sets[1])
            drain_compute(sets[0])
            drain_compute(sets[1])
        else:
            drain_compute(sets[0])
        plsc.subcore_barrier()
        pltpu.sync_copy(acc.at[rows], out_hbm.at[core, rows])

    return edge_kernel


def _proj1(x, wcat, bcat, block_n=1000):
    n, kdim = x.shape
    m = wcat.shape[1]
    d = 128
    return pl.pallas_call(
        _proj1_body,
        grid=(n // block_n,),
        in_specs=[
            pl.BlockSpec((block_n, kdim), lambda i: (i, 0)),
            pl.BlockSpec((kdim, m), lambda i: (0, 0)),
            pl.BlockSpec((1, m), lambda i: (0, 0)),
        ],
        out_specs=[
            pl.BlockSpec((block_n, d), lambda i: (i, 0)),
            pl.BlockSpec((block_n, 2 * d), lambda i: (i, 0)),
            pl.BlockSpec((block_n, d), lambda i: (i, 0)),
            pl.BlockSpec((block_n, 1), lambda i: (i, 0)),
            pl.BlockSpec((block_n, 1), lambda i: (i, 0)),
        ],
        out_shape=[
            jax.ShapeDtypeStruct((n, d), jnp.float32),
            jax.ShapeDtypeStruct((n, 2 * d), jnp.float32),
            jax.ShapeDtypeStruct((n, d), jnp.float32),
            jax.ShapeDtypeStruct((n, 1), jnp.float32),
            jax.ShapeDtypeStruct((n, 1), jnp.float32),
        ],
    )(x, wcat, bcat)


def _proj2(a0, a1, s1, wcat, bcat, block_n=1000):
    n = a0.shape[0]
    aw = a0.shape[1]
    m = wcat.shape[1]
    d = 16
    return pl.pallas_call(
        _proj2_body,
        grid=(n // block_n,),
        in_specs=[
            pl.BlockSpec((block_n, aw), lambda i: (i, 0)),
            pl.BlockSpec((block_n, aw), lambda i: (i, 0)),
            pl.BlockSpec((block_n, 128), lambda i: (i, 0)),
            pl.BlockSpec((128, m), lambda i: (0, 0)),
            pl.BlockSpec((1, m), lambda i: (0, 0)),
        ],
        out_specs=[
            pl.BlockSpec((block_n, d), lambda i: (i, 0)),
            pl.BlockSpec((block_n, 2 * d), lambda i: (i, 0)),
            pl.BlockSpec((block_n, d), lambda i: (i, 0)),
            pl.BlockSpec((block_n, 1), lambda i: (i, 0)),
            pl.BlockSpec((block_n, 1), lambda i: (i, 0)),
        ],
        out_shape=[
            jax.ShapeDtypeStruct((n, d), jnp.float32),
            jax.ShapeDtypeStruct((n, 2 * d), jnp.float32),
            jax.ShapeDtypeStruct((n, d), jnp.float32),
            jax.ShapeDtypeStruct((n, 1), jnp.float32),
            jax.ShapeDtypeStruct((n, 1), jnp.float32),
        ],
    )(a0, a1, s1, wcat, bcat)


def _decode(a0, a1, s2, wl, bl, w2, bb, block_n=1000):
    n = a0.shape[0]
    aw = a0.shape[1]
    g = w2.shape[1]
    rep = jnp.asarray(_REP_NP)
    til = jnp.asarray(_TILE_NP)
    return pl.pallas_call(
        _decode_body,
        grid=(n // block_n,),
        in_specs=[
            pl.BlockSpec((block_n, aw), lambda i: (i, 0)),
            pl.BlockSpec((block_n, aw), lambda i: (i, 0)),
            pl.BlockSpec((block_n, 16), lambda i: (i, 0)),
            pl.BlockSpec((16, 32), lambda i: (0, 0)),
            pl.BlockSpec((1, 32), lambda i: (0, 0)),
            pl.BlockSpec((32, 1024), lambda i: (0, 0)),
            pl.BlockSpec((32, 1024), lambda i: (0, 0)),
            pl.BlockSpec((1024, g), lambda i: (0, 0)),
            pl.BlockSpec((1, g), lambda i: (0, 0)),
        ],
        out_specs=[
            pl.BlockSpec((block_n, g), lambda i: (i, 0)),
            pl.BlockSpec((block_n, 16), lambda i: (i, 0)),
        ],
        out_shape=[
            jax.ShapeDtypeStruct((n, g), jnp.float32),
            jax.ShapeDtypeStruct((n, 16), jnp.float32),
        ],
    )(a0, a1, s2, wl, bl, rep, til, w2, bb)


def _pad_cols(w, to):
    return jnp.pad(w, ((0, 0), (0, to - w.shape[1])))


def kernel(CellX, CellEdgeIndex, Wq1, bq1, Wk1, bk1, Wv1, bv1, Ws1, bs1,
           Wq2, bq2, Wk2, bk2, Wv2, bv2, Ws2, bs2, Wl, bl, Wb, bb):
    n = CellX.shape[0]
    e = CellEdgeIndex.shape[1]
    src = CellEdgeIndex[0]
    dst = CellEdgeIndex[1]
    d1 = Wq1.shape[0]
    d2 = Wq2.shape[0]
    gene = Wb.shape[0]
    adj = Wl.shape[0]

    inv1 = 1.0 / np.sqrt(d1)
    wcat1 = jnp.concatenate([Wq1 * inv1, Wk1, Wv1, Ws1], axis=0).T
    bcat1 = jnp.concatenate([bq1 * inv1, bk1, bv1, bs1])[None, :]
    q1, kv1, s1, qn1, kn1 = _proj1(CellX, wcat1, bcat1)
    m1 = jnp.sqrt(jnp.max(qn1) * jnp.max(kn1))
    mv1 = jnp.full((16,), m1, jnp.float32)

    n_pad = -(-n // (NS * 8)) * (NS * 8)
    ek1 = _make_edge(n, e, d1, 40, 4)
    acc1 = ek1(q1, kv1, src, dst, mv1,
               jnp.zeros((n_pad, d1 + 16), jnp.float32))

    inv2 = 1.0 / np.sqrt(d2)
    wcat2 = jnp.concatenate(
        [_pad_cols((Wq2 * inv2).T, 16), _pad_cols(Wk2.T, 16),
         _pad_cols(Wv2.T, 16), _pad_cols(Ws2.T, 16)], axis=1)
    bcat2 = jnp.concatenate(
        [jnp.pad(bq2 * inv2, (0, 1)), jnp.pad(bk2, (0, 1)),
         jnp.pad(bv2, (0, 1)), jnp.pad(bs2, (0, 1))])[None, :]
    q2, kv2, s2, qn2, kn2 = _proj2(acc1[0, :n], acc1[1, :n],
                                   s1, wcat2, bcat2)
    m2 = jnp.sqrt(jnp.max(qn2) * jnp.max(kn2))
    mv2 = jnp.full((16,), m2, jnp.float32)

    ek2 = _make_edge(n, e, 16, 80, 8)
    acc2 = ek2(q2, kv2, src, dst, mv2,
               jnp.zeros((n_pad, 32), jnp.float32))

    wl_pad = jnp.pad(Wl.T, ((0, 1), (0, 0)))
    w2 = Wb.transpose(1, 2, 0).reshape(adj * adj, gene)
    dec, z_pad = _decode(acc2[0, :n], acc2[1, :n], s2, wl_pad, bl[None, :],
                         w2, bb[None, :])
    return (dec, z_pad[:, :d2])
